# baseline (speedup 1.0000x reference)
import functools

import jax
import jax.numpy as jnp
from jax import lax
from jax.experimental import layout as jexl
from jax.experimental import pallas as pl
from jax.experimental.pallas import tpu as pltpu
from jax.experimental.pallas import tpu_sc as plsc

LEN_TOK = 1000000
VOCAB = LEN_TOK + 1
EMB = 32
BATCH = 4096
SEQ = 200
B = BATCH * SEQ

NC = 2
NS = 16
NW = NC * NS
LANES = 16

B_PER_W = B // NW
CHUNK = 1600
NUM_CHUNKS = B_PER_W // CHUNK

assert B % (8 * NW) == 0
assert B_PER_W % CHUNK == 0 and CHUNK % 8 == 0


def _emb_body(
    tok_hbm, table_hbm, out_hbm,
    idx0, idx1, rows0, rows1, gs0, gs1, ss0, ss1,
):
    wid = lax.axis_index("s") * NC + lax.axis_index("c")
    base = wid * B_PER_W
    idx = (idx0, idx1)
    rows = (rows0, rows1)
    gsem = (gs0, gs1)
    ssem = (ss0, ss1)

    def load_shift(i, ibuf):
        off = base + i * CHUNK
        pltpu.sync_copy(tok_hbm.at[pl.ds(off, CHUNK)], ibuf)

        def shift(j, c):
            sl = pl.ds(j * LANES, LANES)
            ibuf[sl] = jnp.maximum(ibuf[sl] + 1, 0)
            return c

        lax.fori_loop(0, CHUNK // LANES, shift, 0)

    def gather(i, b):
        return pltpu.async_copy(table_hbm.at[idx[b]], rows[b], gsem[b])

    def store(i, b):
        off = base + i * CHUNK
        return pltpu.async_copy(
            rows[b], out_hbm.at[pl.ds(off, CHUNK), pl.ds(0, EMB)], ssem[b]
        )

    gh = [None] * NUM_CHUNKS
    sh = [None] * NUM_CHUNKS
    load_shift(0, idx[0])
    gh[0] = gather(0, 0)
    for i in range(1, NUM_CHUNKS):
        b = i % 2
        if i >= 2:
            sh[i - 2].wait()
        load_shift(i, idx[b])
        gh[i] = gather(i, b)
        gh[i - 1].wait()
        sh[i - 1] = store(i - 1, 1 - b)
    gh[NUM_CHUNKS - 1].wait()
    sh[NUM_CHUNKS - 1] = store(NUM_CHUNKS - 1, (NUM_CHUNKS - 1) % 2)
    sh[NUM_CHUNKS - 2].wait()
    sh[NUM_CHUNKS - 1].wait()


def _impl(tokens, text_emb_weight):
    flat = tokens.reshape(B)
    call = functools.partial(
        pl.kernel,
        mesh=plsc.VectorSubcoreMesh(core_axis_name="c", subcore_axis_name="s"),
        out_type=jax.ShapeDtypeStruct((B, 128), jnp.float32),
        scratch_types=[
            pltpu.VMEM((CHUNK,), jnp.int32),
            pltpu.VMEM((CHUNK,), jnp.int32),
            pltpu.VMEM((CHUNK, EMB), jnp.float32),
            pltpu.VMEM((CHUNK, EMB), jnp.float32),
            pltpu.SemaphoreType.DMA,
            pltpu.SemaphoreType.DMA,
            pltpu.SemaphoreType.DMA,
            pltpu.SemaphoreType.DMA,
        ],
        compiler_params=pltpu.CompilerParams(use_tc_tiling_on_sc=False),
    )(_emb_body)
    out = call(flat, text_emb_weight)
    result = out[:, :EMB].reshape(BATCH, SEQ, EMB)
    return jexl.with_layout_constraint(
        result, jexl.Layout(major_to_minor=(0, 1, 2))
    )


@jax.jit
def kernel(tokens, text_emb_weight):
    return _impl(tokens, text_emb_weight)
